# baseline (speedup 1.0000x reference)
import functools
import jax
import jax.numpy as jnp
from jax import lax
from jax.experimental import pallas as pl
from jax.experimental.pallas import tpu as pltpu
from jax.experimental.pallas import tpu_sc as plsc

_B, _T, _V = 16, 2048, 1000
_HI = 5.0
_LO = -5.0
_NC, _NS, _L = 2, 16, 16
_TRB = _V // 8
_TCB = _T // 128
_VCH = 24
_NTCH = (_VCH // 8) * _TCB
_V0SPLIT = 496
_NCH = 21


def _make_sc_onehot():
    @functools.partial(
        pl.kernel,
        mesh=plsc.VectorSubcoreMesh(core_axis_name="c", subcore_axis_name="s"),
        out_type=jax.ShapeDtypeStruct((_B * _TRB * _TCB, 8, 128), jnp.float32),
        scratch_types=[
            pltpu.VMEM((_T,), jnp.int32),
            pltpu.VMEM((_NTCH, 8, 128), jnp.float32),
            pltpu.VMEM((_NTCH, 8, 128), jnp.float32),
            pltpu.SemaphoreType.DMA,
            pltpu.SemaphoreType.DMA,
        ],
        compiler_params=pltpu.CompilerParams(use_tc_tiling_on_sc=False,
                                             needs_layout_passes=False),
    )
    def _sc_onehot(ids_hbm, out_hbm, ids_v, buf0, buf1, sem0, sem1):
        wid = lax.axis_index("s") * _NC + lax.axis_index("c")
        b = wid // 2
        half = wid % 2
        vbase = half * _V0SPLIT
        pltpu.sync_copy(ids_hbm.at[pl.ds(b * _T, _T)], ids_v)

        lo = jnp.full((_L,), _LO, jnp.float32)
        hi = jnp.full((_L,), _HI, jnp.float32)
        iota = lax.broadcasted_iota(jnp.int32, (_L,), 0)

        def fill(buf):
            def fill_tile(r, carry):
                for s in range(8):
                    for l in range(128 // _L):
                        buf[r, s, pl.ds(l * _L, _L)] = lo
                return carry
            lax.fori_loop(0, _NTCH, fill_tile, 0)

        def poke(buf, v0, nv, val):
            def poke_g(g8, carry):
                for u in range(8):
                    off = (g8 * 8 + u) * _L
                    ids16 = ids_v[pl.ds(off, _L)]
                    vloc = ids16 - v0
                    tile16 = (vloc >> 3) * _TCB + (off >> 7)
                    sub16 = vloc & 7
                    lane16 = (off & 127) + iota
                    mask = vloc.astype(jnp.uint32) < jnp.uint32(nv)
                    plsc.store_scatter(buf, [tile16, sub16, lane16], val,
                                       mask=mask)
                return carry
            lax.fori_loop(0, _T // _L // 8, poke_g, 0)

        def poke2(buf, v_old, nv_old, v_new, nv_new):
            def poke_g(g8, carry):
                for u in range(8):
                    off = (g8 * 8 + u) * _L
                    ids16 = ids_v[pl.ds(off, _L)]
                    lane16 = (off & 127) + iota
                    v_o = ids16 - v_old
                    m_o = v_o.astype(jnp.uint32) < jnp.uint32(nv_old)
                    plsc.store_scatter(
                        buf, [(v_o >> 3) * _TCB + (off >> 7), v_o & 7, lane16],
                        lo, mask=m_o)
                    v_n = ids16 - v_new
                    m_n = v_n.astype(jnp.uint32) < jnp.uint32(nv_new)
                    plsc.store_scatter(
                        buf, [(v_n >> 3) * _TCB + (off >> 7), v_n & 7, lane16],
                        hi, mask=m_n)
                return carry
            lax.fori_loop(0, _T // _L // 8, poke_g, 0)

        def start(buf, sem, k, nv):
            poke(buf, vbase + k * _VCH, nv, hi)
            tile0 = (b * _TRB + (vbase + k * _VCH) // 8) * _TCB
            pltpu.async_copy(buf.at[pl.ds(0, (nv // 8) * _TCB)],
                             out_hbm.at[pl.ds(tile0, (nv // 8) * _TCB)], sem)

        def wait(buf, sem, nv):
            pltpu.make_async_copy(buf.at[pl.ds(0, (nv // 8) * _TCB)],
                                  out_hbm.at[pl.ds(0, (nv // 8) * _TCB)],
                                  sem).wait()

        fill(buf0)
        start(buf0, sem0, 0, _VCH)
        fill(buf1)
        start(buf1, sem1, 1, _VCH)

        def step(buf, sem, k_old, k_new, nv_new):
            wait(buf, sem, _VCH)
            poke2(buf, vbase + k_old * _VCH, _VCH,
                  vbase + k_new * _VCH, nv_new)
            tile0 = (b * _TRB + (vbase + k_new * _VCH) // 8) * _TCB
            pltpu.async_copy(buf.at[pl.ds(0, (nv_new // 8) * _TCB)],
                             out_hbm.at[pl.ds(tile0, (nv_new // 8) * _TCB)],
                             sem)

        def two_body(j, carry):
            k0 = 2 + 2 * j
            step(buf0, sem0, k0 - 2, k0, _VCH)
            step(buf1, sem1, k0 - 1, k0 + 1, _VCH)
            return carry

        lax.fori_loop(0, (_NCH - 3) // 2, two_body, 0)

        @pl.when(half == 0)
        def _():
            step(buf0, sem0, 18, 20, _V0SPLIT - 20 * _VCH)
            wait(buf0, sem0, _V0SPLIT - 20 * _VCH)

        @pl.when(half == 1)
        def _():
            step(buf0, sem0, 18, 20, _VCH)
            wait(buf0, sem0, _VCH)

        wait(buf1, sem1, _VCH)

    return _sc_onehot


def kernel(input_ids):
    Bx, Tx = input_ids.shape
    ids = (input_ids.astype(jnp.int32) % _V).reshape(-1)
    out = _make_sc_onehot()(ids)
    out5 = out.reshape(Bx, _TRB, _TCB, 8, 128)
    q = out5.transpose(0, 1, 3, 2, 4).reshape(Bx, _V, Tx)
    return jnp.swapaxes(q, 1, 2)
